# baseline (speedup 1.0000x reference)
import jax
import jax.numpy as jnp
from jax import lax
from jax.experimental import pallas as pl
from jax.experimental.pallas import tpu as pltpu
from jax.experimental.pallas import tpu_sc as plsc

BATCH = 16384
EMBED_DIM = 32
PACK = 128 // EMBED_DIM

_info = plsc.get_sparse_core_info()
_NC, _NS = _info.num_cores, _info.num_subcores
_NW = _NC * _NS
_B_PER_W = BATCH // _NW
_CHUNK = 128
_N_CHUNKS = _B_PER_W // _CHUNK
_L = 16


def _select(block, mod_v, stage, cb):
    col0 = lax.iota(jnp.int32, _L)
    col1 = col0 + _L

    def grp(g, carry):
        kvec = mod_v[pl.ds(cb + g * _L, _L)] * EMBED_DIM
        for l in range(_L):
            r = g * _L + l
            rsplat = jnp.broadcast_to(r, (_L,)).astype(jnp.int32)
            off = kvec[l]
            stage[r, pl.ds(0, _L)] = plsc.load_gather(
                block, [rsplat, off + col0])
            stage[r, pl.ds(_L, _L)] = plsc.load_gather(
                block, [rsplat, off + col1])
        return carry

    lax.fori_loop(0, _CHUNK // _L, grp, 0)


def _body(di_hbm, mi_hbm, dj_hbm, mj_hbm, emb_hbm, ctx_hbm, ui_hbm, uj_hbm,
          div_i_v, mod_i_v, div_j_v, mod_j_v, bufs, stage, sems):
    wid = lax.axis_index("s") * _NC + lax.axis_index("c")
    base = wid * _B_PER_W
    pltpu.sync_copy(di_hbm.at[pl.ds(base, _B_PER_W)], div_i_v)
    pltpu.sync_copy(mi_hbm.at[pl.ds(base, _B_PER_W)], mod_i_v)
    pltpu.sync_copy(dj_hbm.at[pl.ds(base, _B_PER_W)], div_j_v)
    pltpu.sync_copy(mj_hbm.at[pl.ds(base, _B_PER_W)], mod_j_v)
    tables = (emb_hbm, ctx_hbm)
    divs = (div_i_v, div_j_v)
    mods = (mod_i_v, mod_j_v)
    outs = (ui_hbm, uj_hbm)

    def fire(t, c, slot):
        pltpu.make_async_copy(
            tables[t].at[divs[t].at[pl.ds(c * _CHUNK, _CHUNK)]],
            bufs[slot], sems[slot]).start()

    for t in range(2):
        fire(t, 0, t)
    for c in range(_N_CHUNKS):
        for t in range(2):
            slot = 2 * (c % 2) + t
            nxt = 2 * ((c + 1) % 2) + t
            if c + 1 < _N_CHUNKS:
                fire(t, c + 1, nxt)
            pltpu.make_async_copy(
                tables[t].at[divs[t].at[pl.ds(c * _CHUNK, _CHUNK)]],
                bufs[slot], sems[slot]).wait()
            _select(bufs[slot], mods[t], stage, c * _CHUNK)
            pltpu.sync_copy(stage,
                            outs[t].at[pl.ds(base + c * _CHUNK, _CHUNK)])


def kernel(nodeindex, v_i, v_j, embeddings, context_embeddings):
    del nodeindex
    emb_p = embeddings.reshape(embeddings.shape[0] // PACK, 128)
    ctx_p = context_embeddings.reshape(context_embeddings.shape[0] // PACK,
                                       128)
    vi_div = jax.lax.shift_right_logical(v_i, 2)
    vi_mod = jax.lax.bitwise_and(v_i, PACK - 1)
    vj_div = jax.lax.shift_right_logical(v_j, 2)
    vj_mod = jax.lax.bitwise_and(v_j, PACK - 1)
    mesh = plsc.VectorSubcoreMesh(core_axis_name="c", subcore_axis_name="s")
    k = pl.kernel(
        _body,
        out_type=(
            jax.ShapeDtypeStruct((BATCH, EMBED_DIM), jnp.float32),
            jax.ShapeDtypeStruct((BATCH, EMBED_DIM), jnp.float32),
        ),
        mesh=mesh,
        compiler_params=pltpu.CompilerParams(needs_layout_passes=False),
        scratch_types=[
            pltpu.VMEM((_B_PER_W,), jnp.int32),
            pltpu.VMEM((_B_PER_W,), jnp.int32),
            pltpu.VMEM((_B_PER_W,), jnp.int32),
            pltpu.VMEM((_B_PER_W,), jnp.int32),
            [pltpu.VMEM((_CHUNK, 128), jnp.float32) for _ in range(4)],
            pltpu.VMEM((_CHUNK, EMBED_DIM), jnp.float32),
            [pltpu.SemaphoreType.DMA for _ in range(4)],
        ],
    )
    u_i, u_j = k(vi_div, vi_mod, vj_div, vj_mod, emb_p, ctx_p)
    return (u_i, u_j)


# speedup vs baseline: 1.5066x; 1.5066x over previous
import jax
import jax.numpy as jnp
from jax import lax
from jax.experimental import pallas as pl
from jax.experimental.pallas import tpu as pltpu
from jax.experimental.pallas import tpu_sc as plsc

BATCH = 16384
EMBED_DIM = 32

_info = plsc.get_sparse_core_info()
_NC, _NS = _info.num_cores, _info.num_subcores
_NW = _NC * _NS
_B_PER_W = BATCH // _NW
_CHUNK = 128
_N_CHUNKS = _B_PER_W // _CHUNK
_L = 16


def _fire(table_hbm, idx_v, buf, sem, cb):
    def grp(g, carry):
        vec = idx_v[pl.ds(cb + g * _L, _L)]
        for l in range(_L):
            pltpu.make_async_copy(
                table_hbm.at[vec[l]], buf.at[g * _L + l], sem).start()
        return carry
    lax.fori_loop(0, _CHUNK // _L, grp, 0)


def _drain(table_hbm, buf, sem):
    pltpu.make_async_copy(table_hbm.at[pl.ds(0, _CHUNK)], buf, sem).wait()


def _body(vi_hbm, vj_hbm, emb_hbm, ctx_hbm, ui_hbm, uj_hbm,
          idx_i_v, idx_j_v, bufs, sems):
    wid = lax.axis_index("s") * _NC + lax.axis_index("c")
    base = wid * _B_PER_W
    pltpu.sync_copy(vi_hbm.at[pl.ds(base, _B_PER_W)], idx_i_v)
    pltpu.sync_copy(vj_hbm.at[pl.ds(base, _B_PER_W)], idx_j_v)
    tables = (emb_hbm, ctx_hbm)
    idxs = (idx_i_v, idx_j_v)
    outs = (ui_hbm, uj_hbm)
    for t in range(2):
        for c in range(2):
            _fire(tables[t], idxs[t], bufs[2 * c + t], sems[2 * c + t],
                  c * _CHUNK)
    for c in range(_N_CHUNKS):
        for t in range(2):
            slot = 2 * (c % 2) + t
            _drain(tables[t], bufs[slot], sems[slot])
            pltpu.sync_copy(bufs[slot],
                            outs[t].at[pl.ds(base + c * _CHUNK, _CHUNK)])
            if c + 2 < _N_CHUNKS:
                _fire(tables[t], idxs[t], bufs[slot], sems[slot],
                      (c + 2) * _CHUNK)


def kernel(nodeindex, v_i, v_j, embeddings, context_embeddings):
    del nodeindex
    mesh = plsc.VectorSubcoreMesh(core_axis_name="c", subcore_axis_name="s")
    k = pl.kernel(
        _body,
        out_type=(
            jax.ShapeDtypeStruct((BATCH, EMBED_DIM), jnp.float32),
            jax.ShapeDtypeStruct((BATCH, EMBED_DIM), jnp.float32),
        ),
        mesh=mesh,
        scratch_types=[
            pltpu.VMEM((_B_PER_W,), jnp.int32),
            pltpu.VMEM((_B_PER_W,), jnp.int32),
            [pltpu.VMEM((_CHUNK, EMBED_DIM), jnp.float32) for _ in range(4)],
            [pltpu.SemaphoreType.DMA for _ in range(4)],
        ],
    )
    u_i, u_j = k(v_i, v_j, embeddings, context_embeddings)
    return (u_i, u_j)
